# baseline (speedup 1.0000x reference)
import functools

import jax
import jax.numpy as jnp
from jax import lax
from jax.experimental import pallas as pl
from jax.experimental.pallas import tpu as pltpu
from jax.experimental.pallas import tpu_sc as plsc

BATCH = 16384
HIST = 200
DIM = 16
N_VOCAB = 1000000

NUM_CORES = 2
NUM_SUBCORES = 16
NW = NUM_CORES * NUM_SUBCORES
ITEMS_PER_W = BATCH // NW
CHUNK_ITEMS = 16
CHUNK_ROWS = CHUNK_ITEMS * HIST
STREAM = 128
N_STREAMS = CHUNK_ROWS // STREAM
N_CHUNKS = ITEMS_PER_W // CHUNK_ITEMS
X2_ROWS_PER_CHUNK = CHUNK_ROWS // STREAM
X2_ROWS_PER_W = ITEMS_PER_W * HIST // STREAM


def _body(x_hbm, tab_hbm, out_hbm, idx0, idx1, rows0, rows1, acc0, acc1,
          sem0, sem1, isem0, isem1, osem0, osem1):
    wid = lax.axis_index("s") * NUM_CORES + lax.axis_index("c")
    inv = jnp.float32(1.0 / HIST)
    idx_b = (idx0, idx1)
    rows_b = (rows0, rows1)
    acc_b = (acc0, acc1)
    sem_b = (sem0, sem1)
    isem_b = (isem0, isem1)
    osem_b = (osem0, osem1)

    def streams(b):
        out = []
        for c in range(CHUNK_ITEMS):
            for off, n in ((0, 120), (120, 80)):
                out.append(
                    (idx_b[b].at[c, pl.ds(off, n)],
                     rows_b[b].at[pl.ds(c * HIST + off, n)])
                )
        return out

    def xsrc(k):
        row0 = wid * ITEMS_PER_W + k * CHUNK_ITEMS
        return x_hbm.at[pl.ds(row0, CHUNK_ITEMS), :]

    def fire_idx(k, b):
        pltpu.async_copy(xsrc(k), idx_b[b], isem_b[b])

    def drain_idx(k, b):
        pltpu.make_async_copy(xsrc(k), idx_b[b], isem_b[b]).wait()

    def fire_gathers(b):
        for isl, rsl in streams(b):
            pltpu.async_copy(tab_hbm.at[isl], rsl, sem_b[b])

    def drain_gathers(b):
        for isl, rsl in streams(b):
            pltpu.make_async_copy(tab_hbm.at[isl], rsl, sem_b[b]).wait()

    def odst(k):
        row0 = wid * ITEMS_PER_W + k * CHUNK_ITEMS
        return out_hbm.at[pl.ds(row0, CHUNK_ITEMS), :]

    def accumulate(k, b):
        rows_v = rows_b[b]
        acc_v = acc_b[b]
        per_item = list(zip(*[iter(streams(b))] * 2))

        for c in range(CHUNK_ITEMS):
            for isl, rsl in per_item[c]:
                pltpu.make_async_copy(tab_hbm.at[isl], rsl, sem_b[b]).wait()
            base = c * HIST

            def j_body(j, acc):
                return acc + rows_v[base + j, :]

            acc = lax.fori_loop(
                0, HIST, j_body, jnp.zeros((DIM,), jnp.float32), unroll=25
            )
            acc_v[c, :] = acc * inv

        pltpu.async_copy(acc_v, odst(k), osem_b[b])

    def drain_out(k, b):
        pltpu.make_async_copy(acc_b[b], odst(k), osem_b[b]).wait()

    fire_idx(0, 0)
    drain_idx(0, 0)
    fire_gathers(0)
    fire_idx(1, 1)
    drain_idx(1, 1)

    def outer_body(kk, carry):
        for b in (0, 1):
            k = 2 * kk + b
            fire_gathers(1 - b)
            fire_idx(k + 2, b)
            pl.when(k >= 2)(lambda: drain_out(k - 2, b))
            accumulate(k, b)
            drain_idx(k + 2, b)
        return carry

    lax.fori_loop(0, (N_CHUNKS - 2) // 2, outer_body, 0)
    k = N_CHUNKS - 2
    fire_gathers(1)
    drain_out(k - 2, 0)
    accumulate(k, 0)
    drain_out(k - 1, 1)
    accumulate(k + 1, 1)
    drain_out(k, 0)
    drain_out(k + 1, 1)


@jax.jit
def kernel(x, embed_table):
    mesh = plsc.VectorSubcoreMesh(core_axis_name="c", subcore_axis_name="s")
    run = pl.kernel(
        _body,
        out_type=jax.ShapeDtypeStruct((BATCH, DIM), jnp.float32),
        mesh=mesh,
        scratch_types=[
            pltpu.VMEM((CHUNK_ITEMS, HIST), jnp.int32),
            pltpu.VMEM((CHUNK_ITEMS, HIST), jnp.int32),
            pltpu.VMEM((CHUNK_ROWS, DIM), jnp.float32),
            pltpu.VMEM((CHUNK_ROWS, DIM), jnp.float32),
            pltpu.VMEM((CHUNK_ITEMS, DIM), jnp.float32),
            pltpu.VMEM((CHUNK_ITEMS, DIM), jnp.float32),
            pltpu.SemaphoreType.DMA,
            pltpu.SemaphoreType.DMA,
            pltpu.SemaphoreType.DMA,
            pltpu.SemaphoreType.DMA,
            pltpu.SemaphoreType.DMA,
            pltpu.SemaphoreType.DMA,
        ],
        compiler_params=pltpu.CompilerParams(use_tc_tiling_on_sc=False),
    )
    return run(x, embed_table)


# speedup vs baseline: 1.0406x; 1.0406x over previous
import functools

import jax
import jax.numpy as jnp
from jax import lax
from jax.experimental import pallas as pl
from jax.experimental.pallas import tpu as pltpu
from jax.experimental.pallas import tpu_sc as plsc

BATCH = 16384
HIST = 200
DIM = 16
N_VOCAB = 1000000

NUM_CORES = 2
NUM_SUBCORES = 16
NW = NUM_CORES * NUM_SUBCORES
ITEMS_PER_W = BATCH // NW
CHUNK_ITEMS = 16
CHUNK_ROWS = CHUNK_ITEMS * HIST
STREAM = 128
N_STREAMS = CHUNK_ROWS // STREAM
N_CHUNKS = ITEMS_PER_W // CHUNK_ITEMS
X2_ROWS_PER_CHUNK = CHUNK_ROWS // STREAM
X2_ROWS_PER_W = ITEMS_PER_W * HIST // STREAM


def _body(x_hbm, tab_hbm, out_hbm, idx0, idx1, rows0, rows1, acc0, acc1,
          sem0, sem1, isem0, isem1, osem0, osem1):
    wid = lax.axis_index("s") * NUM_CORES + lax.axis_index("c")
    inv = jnp.float32(1.0 / HIST)
    idx_b = (idx0, idx1)
    rows_b = (rows0, rows1)
    acc_b = (acc0, acc1)
    sem_b = (sem0, sem1)
    isem_b = (isem0, isem1)
    osem_b = (osem0, osem1)

    def streams(b):
        out = []
        for c in range(CHUNK_ITEMS):
            for off, n in ((0, 120), (120, 80)):
                out.append(
                    (idx_b[b].at[c, pl.ds(off, n)],
                     rows_b[b].at[pl.ds(c * HIST + off, n)])
                )
        return out

    def xsrc(k):
        row0 = wid * ITEMS_PER_W + k * CHUNK_ITEMS
        return x_hbm.at[pl.ds(row0, CHUNK_ITEMS), :]

    def fire_idx(k, b):
        pltpu.async_copy(xsrc(k), idx_b[b], isem_b[b])

    def drain_idx(k, b):
        pltpu.make_async_copy(xsrc(k), idx_b[b], isem_b[b]).wait()

    def fire_gathers(b):
        for isl, rsl in streams(b):
            pltpu.async_copy(tab_hbm.at[isl], rsl, sem_b[b])

    def drain_gathers(b):
        for isl, rsl in streams(b):
            pltpu.make_async_copy(tab_hbm.at[isl], rsl, sem_b[b]).wait()

    def odst(k):
        row0 = wid * ITEMS_PER_W + k * CHUNK_ITEMS
        return out_hbm.at[pl.ds(row0, CHUNK_ITEMS), :]

    def accumulate(k, b):
        rows_v = rows_b[b]
        acc_v = acc_b[b]
        per_item = list(zip(*[iter(streams(b))] * 2))

        for c in range(CHUNK_ITEMS):
            for isl, rsl in per_item[c]:
                pltpu.make_async_copy(tab_hbm.at[isl], rsl, sem_b[b]).wait()
            base = c * HIST

            def j_body(j, acc):
                return acc + rows_v[base + j, :]

            acc = lax.fori_loop(
                0, HIST, j_body, jnp.zeros((DIM,), jnp.float32), unroll=10
            )
            acc_v[c, :] = acc * inv

        pltpu.async_copy(acc_v, odst(k), osem_b[b])

    def drain_out(k, b):
        pltpu.make_async_copy(acc_b[b], odst(k), osem_b[b]).wait()

    fire_idx(0, 0)
    drain_idx(0, 0)
    fire_gathers(0)
    fire_idx(1, 1)
    drain_idx(1, 1)

    def outer_body(kk, carry):
        for b in (0, 1):
            k = 2 * kk + b
            fire_gathers(1 - b)
            fire_idx(k + 2, b)
            pl.when(k >= 2)(lambda: drain_out(k - 2, b))
            accumulate(k, b)
            drain_idx(k + 2, b)
        return carry

    lax.fori_loop(0, (N_CHUNKS - 2) // 2, outer_body, 0)
    k = N_CHUNKS - 2
    fire_gathers(1)
    drain_out(k - 2, 0)
    accumulate(k, 0)
    drain_out(k - 1, 1)
    accumulate(k + 1, 1)
    drain_out(k, 0)
    drain_out(k + 1, 1)


@jax.jit
def kernel(x, embed_table):
    mesh = plsc.VectorSubcoreMesh(core_axis_name="c", subcore_axis_name="s")
    run = pl.kernel(
        _body,
        out_type=jax.ShapeDtypeStruct((BATCH, DIM), jnp.float32),
        mesh=mesh,
        scratch_types=[
            pltpu.VMEM((CHUNK_ITEMS, HIST), jnp.int32),
            pltpu.VMEM((CHUNK_ITEMS, HIST), jnp.int32),
            pltpu.VMEM((CHUNK_ROWS, DIM), jnp.float32),
            pltpu.VMEM((CHUNK_ROWS, DIM), jnp.float32),
            pltpu.VMEM((CHUNK_ITEMS, DIM), jnp.float32),
            pltpu.VMEM((CHUNK_ITEMS, DIM), jnp.float32),
            pltpu.SemaphoreType.DMA,
            pltpu.SemaphoreType.DMA,
            pltpu.SemaphoreType.DMA,
            pltpu.SemaphoreType.DMA,
            pltpu.SemaphoreType.DMA,
            pltpu.SemaphoreType.DMA,
        ],
        compiler_params=pltpu.CompilerParams(use_tc_tiling_on_sc=False),
    )
    return run(x, embed_table)


# speedup vs baseline: 1.0472x; 1.0063x over previous
import functools

import jax
import jax.numpy as jnp
from jax import lax
from jax.experimental import pallas as pl
from jax.experimental.pallas import tpu as pltpu
from jax.experimental.pallas import tpu_sc as plsc

BATCH = 16384
HIST = 200
DIM = 16
N_VOCAB = 1000000

NUM_CORES = 2
NUM_SUBCORES = 16
NW = NUM_CORES * NUM_SUBCORES
ITEMS_PER_W = BATCH // NW
CHUNK_ITEMS = 16
CHUNK_ROWS = CHUNK_ITEMS * HIST
STREAM = 128
N_STREAMS = CHUNK_ROWS // STREAM
N_CHUNKS = ITEMS_PER_W // CHUNK_ITEMS
X2_ROWS_PER_CHUNK = CHUNK_ROWS // STREAM
X2_ROWS_PER_W = ITEMS_PER_W * HIST // STREAM


def _body(x_hbm, tab_hbm, out_hbm, idx0, idx1, rows0, rows1, acc0, acc1,
          sem0, sem1, isem0, isem1, osem0, osem1):
    wid = lax.axis_index("s") * NUM_CORES + lax.axis_index("c")
    inv = jnp.float32(1.0 / HIST)
    idx_b = (idx0, idx1)
    rows_b = (rows0, rows1)
    acc_b = (acc0, acc1)
    sem_b = (sem0, sem1)
    isem_b = (isem0, isem1)
    osem_b = (osem0, osem1)

    def streams(b):
        out = []
        for c in range(CHUNK_ITEMS):
            for off, n in ((0, 120), (120, 80)):
                out.append(
                    (idx_b[b].at[c, pl.ds(off, n)],
                     rows_b[b].at[pl.ds(c * HIST + off, n)])
                )
        return out

    def xsrc(k):
        row0 = wid * ITEMS_PER_W + k * CHUNK_ITEMS
        return x_hbm.at[pl.ds(row0, CHUNK_ITEMS), :]

    def fire_idx(k, b):
        pltpu.async_copy(xsrc(k), idx_b[b], isem_b[b])

    def drain_idx(k, b):
        pltpu.make_async_copy(xsrc(k), idx_b[b], isem_b[b]).wait()

    def fire_gathers(b):
        for isl, rsl in streams(b):
            pltpu.async_copy(tab_hbm.at[isl], rsl, sem_b[b])

    def drain_gathers(b):
        for isl, rsl in streams(b):
            pltpu.make_async_copy(tab_hbm.at[isl], rsl, sem_b[b]).wait()

    def odst(k):
        row0 = wid * ITEMS_PER_W + k * CHUNK_ITEMS
        return out_hbm.at[pl.ds(row0, CHUNK_ITEMS), :]

    def accumulate(k, b):
        rows_v = rows_b[b]
        acc_v = acc_b[b]
        per_item = list(zip(*[iter(streams(b))] * 2))

        for c in range(CHUNK_ITEMS):
            for isl, rsl in per_item[c]:
                pltpu.make_async_copy(tab_hbm.at[isl], rsl, sem_b[b]).wait()
            base = c * HIST

            def j_body(j, acc):
                return acc + rows_v[base + j, :]

            acc = lax.fori_loop(
                0, HIST, j_body, jnp.zeros((DIM,), jnp.float32), unroll=8
            )
            acc_v[c, :] = acc * inv

        pltpu.async_copy(acc_v, odst(k), osem_b[b])

    def drain_out(k, b):
        pltpu.make_async_copy(acc_b[b], odst(k), osem_b[b]).wait()

    fire_idx(0, 0)
    drain_idx(0, 0)
    fire_gathers(0)
    fire_idx(1, 1)
    drain_idx(1, 1)

    def outer_body(kk, carry):
        for b in (0, 1):
            k = 2 * kk + b
            fire_gathers(1 - b)
            fire_idx(k + 2, b)
            pl.when(k >= 2)(lambda: drain_out(k - 2, b))
            accumulate(k, b)
            drain_idx(k + 2, b)
        return carry

    lax.fori_loop(0, (N_CHUNKS - 2) // 2, outer_body, 0)
    k = N_CHUNKS - 2
    fire_gathers(1)
    drain_out(k - 2, 0)
    accumulate(k, 0)
    drain_out(k - 1, 1)
    accumulate(k + 1, 1)
    drain_out(k, 0)
    drain_out(k + 1, 1)


@jax.jit
def kernel(x, embed_table):
    mesh = plsc.VectorSubcoreMesh(core_axis_name="c", subcore_axis_name="s")
    run = pl.kernel(
        _body,
        out_type=jax.ShapeDtypeStruct((BATCH, DIM), jnp.float32),
        mesh=mesh,
        scratch_types=[
            pltpu.VMEM((CHUNK_ITEMS, HIST), jnp.int32),
            pltpu.VMEM((CHUNK_ITEMS, HIST), jnp.int32),
            pltpu.VMEM((CHUNK_ROWS, DIM), jnp.float32),
            pltpu.VMEM((CHUNK_ROWS, DIM), jnp.float32),
            pltpu.VMEM((CHUNK_ITEMS, DIM), jnp.float32),
            pltpu.VMEM((CHUNK_ITEMS, DIM), jnp.float32),
            pltpu.SemaphoreType.DMA,
            pltpu.SemaphoreType.DMA,
            pltpu.SemaphoreType.DMA,
            pltpu.SemaphoreType.DMA,
            pltpu.SemaphoreType.DMA,
            pltpu.SemaphoreType.DMA,
        ],
        compiler_params=pltpu.CompilerParams(use_tc_tiling_on_sc=False),
    )
    return run(x, embed_table)
